# baseline (speedup 1.0000x reference)
import functools

import jax
import jax.numpy as jnp
from jax import lax
from jax.experimental import pallas as pl
from jax.experimental.pallas import tpu as pltpu
from jax.experimental.pallas import tpu_sc as plsc

_E = 8
_K = 2
_D = 1024
_T = 2048
_S = _T * _K

_NC = 2
_NS = 16
_NW = _NC * _NS
_TPW = _T // _NW
_SPW = _S // _NW

_TM = 512
_NT = _S // _TM
_NL = _NT + _E - 1


def _wid():
    return lax.axis_index("s") * _NC + lax.axis_index("c")


def _dispatch_body(hs_hbm, pe_hbm, po_hbm, we_hbm, wo_hbm, xs_hbm, ws_hbm,
                   idx_e, idx_o, xin, wv_e, wv_o, s_e, s_o, s_x, s_w):
    w = _wid()
    base_t = w * _TPW
    ce = pltpu.async_copy(pe_hbm.at[pl.ds(base_t, _TPW)], idx_e, s_e)
    co = pltpu.async_copy(po_hbm.at[pl.ds(base_t, _TPW)], idx_o, s_o)
    cx = pltpu.async_copy(hs_hbm.at[pl.ds(base_t, _TPW)], xin, s_x)
    cwe = pltpu.async_copy(we_hbm.at[pl.ds(base_t, _TPW)], wv_e, s_w)
    cwo = pltpu.async_copy(wo_hbm.at[pl.ds(base_t, _TPW)], wv_o, s_w)
    ce.wait()
    cx.wait()
    c1 = pltpu.async_copy(xin, xs_hbm.at[idx_e], s_e)
    co.wait()
    c2 = pltpu.async_copy(xin, xs_hbm.at[idx_o], s_o)
    cwe.wait()
    cwo.wait()
    c1.wait()
    c3 = pltpu.async_copy(wv_e, ws_hbm.at[idx_e], s_e)
    c2.wait()
    c4 = pltpu.async_copy(wv_o, ws_hbm.at[idx_o], s_o)
    c3.wait()
    c4.wait()


def _dispatch_sc(hs_bf, pe, po, we, wo):
    mesh = plsc.VectorSubcoreMesh(
        core_axis_name="c", subcore_axis_name="s",
        num_cores=_NC, num_subcores=_NS)
    f = pl.kernel(
        _dispatch_body,
        out_type=(
            jax.ShapeDtypeStruct((_S, _D // 2), jnp.int32),
            jax.ShapeDtypeStruct((_S,), jnp.float32),
        ),
        mesh=mesh,
        scratch_types=(
            pltpu.VMEM((_TPW,), jnp.int32),
            pltpu.VMEM((_TPW,), jnp.int32),
            pltpu.VMEM((_TPW, _D // 2), jnp.int32),
            pltpu.VMEM((_TPW,), jnp.float32),
            pltpu.VMEM((_TPW,), jnp.float32),
            pltpu.SemaphoreType.DMA,
            pltpu.SemaphoreType.DMA,
            pltpu.SemaphoreType.DMA,
            pltpu.SemaphoreType.DMA,
        ),
    )
    return f(hs_bf, pe, po, we, wo)


_CC = 16
_NCH = _TPW // _CC


def _combine_body(y_hbm, pe_hbm, po_hbm, fin_hbm,
                  ies, ios, bufs_e, bufs_o, s_i, s_e, s_o, s_w):
    w = _wid()
    base_t = w * _TPW

    for c in range(_NCH):
        pltpu.async_copy(pe_hbm.at[pl.ds(base_t + c * _CC, _CC)], ies[c], s_i)
        pltpu.async_copy(po_hbm.at[pl.ds(base_t + c * _CC, _CC)], ios[c], s_i)
    for c in range(_NCH):
        pltpu.make_async_copy(pe_hbm.at[pl.ds(base_t, _CC)], ies[c], s_i).wait()
        pltpu.make_async_copy(po_hbm.at[pl.ds(base_t, _CC)], ios[c], s_i).wait()

    ge = [None] * _NCH
    go = [None] * _NCH
    gw = [None] * _NCH
    ge[0] = pltpu.async_copy(y_hbm.at[ies[0]], bufs_e[0], s_e[0])
    go[0] = pltpu.async_copy(y_hbm.at[ios[0]], bufs_o[0], s_o[0])
    for c in range(_NCH):
        nxt = c + 1
        if nxt < _NCH:
            b = nxt % 2
            if nxt >= 2:
                gw[nxt - 2].wait()
            ge[nxt] = pltpu.async_copy(y_hbm.at[ies[nxt]], bufs_e[b], s_e[b])
            go[nxt] = pltpu.async_copy(y_hbm.at[ios[nxt]], bufs_o[b], s_o[b])
        ge[c].wait()
        go[c].wait()
        be, bo = bufs_e[c % 2], bufs_o[c % 2]

        def row_add(r, _):
            def col_add(q, _):
                for u in range(8):
                    o = q * 128 + u * 16
                    be[r, pl.ds(o, 16)] = (
                        be[r, pl.ds(o, 16)] + bo[r, pl.ds(o, 16)])
                return 0
            lax.fori_loop(0, _D // 128, col_add, 0)
            return 0

        lax.fori_loop(0, _CC, row_add, 0)
        gw[c] = pltpu.async_copy(
            be, fin_hbm.at[pl.ds(base_t + c * _CC, _CC)], s_w[c % 2])
    gw[_NCH - 2].wait()
    gw[_NCH - 1].wait()


def _combine_sc(y, pe, po):
    mesh = plsc.VectorSubcoreMesh(
        core_axis_name="c", subcore_axis_name="s",
        num_cores=_NC, num_subcores=_NS)
    f = pl.kernel(
        _combine_body,
        out_type=jax.ShapeDtypeStruct((_T, _D), jnp.float32),
        mesh=mesh,
        scratch_types=(
            tuple(pltpu.VMEM((_CC,), jnp.int32) for _ in range(_NCH)),
            tuple(pltpu.VMEM((_CC,), jnp.int32) for _ in range(_NCH)),
            tuple(pltpu.VMEM((_CC, _D), jnp.float32) for _ in range(2)),
            tuple(pltpu.VMEM((_CC, _D), jnp.float32) for _ in range(2)),
            pltpu.SemaphoreType.DMA,
            tuple(pltpu.SemaphoreType.DMA for _ in range(2)),
            tuple(pltpu.SemaphoreType.DMA for _ in range(2)),
            tuple(pltpu.SemaphoreType.DMA for _ in range(2)),
        ),
    )
    return f(y, pe, po)


def _gmm_body(mt, wi, rs, re, fi, x_ref, w_ref, ws_ref, o_ref):
    i = pl.program_id(0)
    part = lax.dot_general(
        x_ref[...], w_ref[...], (((1,), (1,)), ((), ())),
        preferred_element_type=jnp.float32)
    rows = mt[i] * _TM + lax.broadcasted_iota(jnp.int32, (_TM, 1), 0)
    msk = (rows >= rs[i]) & (rows < re[i])
    val = jnp.where(msk, part * ws_ref[...], 0.0)

    @pl.when(fi[i] == 1)
    def _init():
        o_ref[...] = val

    @pl.when(fi[i] == 0)
    def _acc():
        o_ref[...] += val


def _grouped_matmul(xs, w_bf, ws2, mt, wi, rs, re, fi):
    grid_spec = pltpu.PrefetchScalarGridSpec(
        num_scalar_prefetch=5,
        grid=(_NL,),
        in_specs=[
            pl.BlockSpec((_TM, _D), lambda i, mt, wi, rs, re, fi: (mt[i], 0)),
            pl.BlockSpec((None, _D, _D),
                         lambda i, mt, wi, rs, re, fi: (wi[i], 0, 0)),
            pl.BlockSpec((_TM, 1), lambda i, mt, wi, rs, re, fi: (mt[i], 0)),
        ],
        out_specs=pl.BlockSpec((_TM, _D),
                               lambda i, mt, wi, rs, re, fi: (mt[i], 0)),
    )
    return pl.pallas_call(
        _gmm_body,
        grid_spec=grid_spec,
        out_shape=jax.ShapeDtypeStruct((_S, _D), jnp.float32),
    )(mt, wi, rs, re, fi, xs, w_bf, ws2)


def kernel(hidden_states, gate_w, expert_w):
    B, Sq, D = hidden_states.shape
    hs = hidden_states.reshape(-1, D)

    router_logits = hs @ gate_w.T
    probs = jax.nn.softmax(router_logits.astype(jnp.float32), axis=1)
    m1 = jnp.max(probs, axis=1)
    e1 = jnp.argmax(probs, axis=1).astype(jnp.int32)
    probs2 = jnp.where(jnp.arange(_E)[None, :] == e1[:, None], -jnp.inf, probs)
    m2 = jnp.max(probs2, axis=1)
    e2 = jnp.argmax(probs2, axis=1).astype(jnp.int32)
    denom = m1 + m2
    rw = jnp.stack([m1 / denom, m2 / denom], axis=1)
    sel = jnp.stack([e1, e2], axis=1)

    slot_e = sel.reshape(-1)
    onehot = (slot_e[:, None] == jnp.arange(_E)[None, :]).astype(jnp.float32)
    _CH = 128
    nch = _S // _CH
    oh3 = onehot.reshape(nch, _CH, _E)
    tri = (jnp.arange(_CH)[:, None] >= jnp.arange(_CH)[None, :]).astype(
        jnp.float32)
    hp = lax.Precision.HIGHEST
    csum_in = jnp.einsum("ij,cje->cie", tri, oh3, precision=hp)
    chunk_tot = csum_in[:, -1, :]
    tri_s = (jnp.arange(nch)[:, None] > jnp.arange(nch)[None, :]).astype(
        jnp.float32)
    chunk_base = jnp.matmul(tri_s, chunk_tot, precision=hp)
    csum = (csum_in + chunk_base[:, None, :]).reshape(_S, _E)
    counts = chunk_base[-1] + chunk_tot[-1]
    tri_e = (jnp.arange(_E)[:, None] > jnp.arange(_E)[None, :]).astype(
        jnp.float32)
    offs_x = jnp.matmul(tri_e, counts, precision=hp)
    rank = jnp.sum(onehot * csum, axis=1) - 1.0
    base = jnp.sum(onehot * offs_x[None, :], axis=1)
    pos = (base + rank).astype(jnp.int32)
    offs = jnp.concatenate(
        [offs_x, (offs_x[-1] + counts[-1])[None]]).astype(jnp.int32)

    starts, ends = offs[:_E], offs[1:]
    m_ids = jnp.arange(_NT, dtype=jnp.int32)
    lo = jnp.maximum(starts[None, :], (m_ids * _TM)[:, None])
    hi = jnp.minimum(ends[None, :], ((m_ids + 1) * _TM)[:, None])
    valid = (lo < hi).reshape(-1)
    order = jnp.cumsum(valid.astype(jnp.int32)) - 1
    sel_mat = ((order[None, :] == jnp.arange(_NL)[:, None]) &
               valid[None, :]).astype(jnp.float32)
    m_flat = jnp.broadcast_to(m_ids[:, None], (_NT, _E)).reshape(-1)
    e_flat = jnp.broadcast_to(jnp.arange(_E, dtype=jnp.int32)[None, :],
                              (_NT, _E)).reshape(-1)
    any_row = jnp.sum(sel_mat, axis=1)
    mt = (jnp.matmul(sel_mat, m_flat.astype(jnp.float32), precision=hp)
          + (1.0 - any_row) * (_NT - 1)).astype(jnp.int32)
    wi = jnp.matmul(sel_mat, e_flat.astype(jnp.float32),
                    precision=hp).astype(jnp.int32)
    rs = jnp.matmul(sel_mat, lo.reshape(-1).astype(jnp.float32),
                    precision=hp).astype(jnp.int32)
    re = jnp.matmul(sel_mat, hi.reshape(-1).astype(jnp.float32),
                    precision=hp).astype(jnp.int32)
    fi = jnp.concatenate([jnp.ones((1,), jnp.int32),
                          (mt[1:] != mt[:-1]).astype(jnp.int32)])

    pos2 = pos.reshape(_T, _K)
    pe, po = pos2[:, 0], pos2[:, 1]

    hs_i = lax.bitcast_convert_type(
        hs.astype(jnp.bfloat16).reshape(_T, _D // 2, 2), jnp.int32)
    xs_i, ws = _dispatch_sc(hs_i, pe, po,
                            rw[:, 0].astype(jnp.float32),
                            rw[:, 1].astype(jnp.float32))
    xs = lax.bitcast_convert_type(xs_i, jnp.bfloat16).reshape(_S, _D)
    w_bf = expert_w.astype(jnp.bfloat16)
    y = _grouped_matmul(xs, w_bf, ws.reshape(_S, 1), mt, wi, rs, re, fi)
    fin = _combine_sc(y, pe, po)

    return fin.reshape(B, Sq, D), router_logits


# speedup vs baseline: 3.0277x; 3.0277x over previous
import functools

import jax
import jax.numpy as jnp
from jax import lax
from jax.experimental import pallas as pl
from jax.experimental.pallas import tpu as pltpu
from jax.experimental.pallas import tpu_sc as plsc

_E = 8
_K = 2
_D = 1024
_T = 2048
_S = _T * _K

_NC = 2
_NS = 16
_NW = _NC * _NS
_TPW = _T // _NW
_SPW = _S // _NW

_TM = 512
_NT = _S // _TM
_NL = _NT + _E - 1


def _wid():
    return lax.axis_index("s") * _NC + lax.axis_index("c")


def _dense_moe_body(x_ref, w_ref, wc_ref, o_ref):
    e = pl.program_id(0)
    part = lax.dot_general(
        x_ref[...], w_ref[...],
        (((1,), (1,)), ((), ())),
        preferred_element_type=jnp.float32,
    )
    val = part * wc_ref[...]

    @pl.when(e == 0)
    def _init():
        o_ref[...] = val

    @pl.when(e > 0)
    def _acc():
        o_ref[...] += val


def kernel(hidden_states, gate_w, expert_w):
    B, Sq, D = hidden_states.shape
    hs = hidden_states.reshape(-1, D)

    router_logits = hs @ gate_w.T
    probs = jax.nn.softmax(router_logits.astype(jnp.float32), axis=1)
    m1 = jnp.max(probs, axis=1)
    e1 = jnp.argmax(probs, axis=1).astype(jnp.int32)
    probs2 = jnp.where(jnp.arange(_E)[None, :] == e1[:, None], -jnp.inf, probs)
    m2 = jnp.max(probs2, axis=1)
    e2 = jnp.argmax(probs2, axis=1).astype(jnp.int32)
    denom = m1 + m2
    w1 = m1 / denom
    w2 = m2 / denom
    eye = jnp.arange(_E, dtype=jnp.int32)[None, :]
    wmat = (w1[:, None] * (eye == e1[:, None]) +
            w2[:, None] * (eye == e2[:, None])).astype(jnp.float32)

    x_bf = hs.astype(jnp.bfloat16)
    w_bf = expert_w.astype(jnp.bfloat16)
    wcol = wmat.T.reshape(_E, _T, 1)

    out = pl.pallas_call(
        _dense_moe_body,
        grid=(_E,),
        in_specs=[
            pl.BlockSpec((_T, D), lambda e: (0, 0)),
            pl.BlockSpec((None, D, D), lambda e: (e, 0, 0)),
            pl.BlockSpec((None, _T, 1), lambda e: (e, 0, 0)),
        ],
        out_specs=pl.BlockSpec((_T, D), lambda e: (0, 0)),
        out_shape=jax.ShapeDtypeStruct((_T, D), jnp.float32),
    )(x_bf, w_bf, wcol)

    return out.reshape(B, Sq, D), router_logits
